# Initial kernel scaffold; baseline (speedup 1.0000x reference)
import jax
import jax.numpy as jnp
from jax.experimental import pallas as pl


def kernel(x):
    raise NotImplementedError("write your pallas kernel here")


# speedup vs baseline: 49.3227x; 49.3227x over previous
import functools

import jax
import jax.numpy as jnp
from jax import lax
from jax.experimental import pallas as pl
from jax.experimental.pallas import tpu as pltpu
from jax.experimental.pallas import tpu_sc as plsc

_B, _C, _N = 4, 768, 8192
_K = 8
_ROWS = _B * _C
_NW = 32
_RPW = _ROWS // _NW
_LANES = 16
_VPR = _N // _LANES
_UNROLL = 4


def _vsort_desc(v):
    return plsc.sort_key_val(v, v, descending=True)[0]


def _combine(a, b, lane_lt8):
    return _vsort_desc(jnp.where(lane_lt8, a, lax.rev(b, (0,))))


def _insert(ts, x):
    out = []
    cur = x
    for t in ts:
        out.append(jnp.maximum(t, cur))
        cur = jnp.minimum(t, cur)
    return tuple(out)


@functools.partial(
    pl.kernel,
    out_type=jax.ShapeDtypeStruct((_ROWS * _K,), jnp.float32),
    mesh=plsc.VectorSubcoreMesh(core_axis_name="c", subcore_axis_name="s"),
    scratch_types=[
        pltpu.VMEM((_N,), jnp.float32),
        pltpu.VMEM((_N,), jnp.float32),
        pltpu.VMEM((_RPW * _K + _LANES - _K,), jnp.float32),
        pltpu.SemaphoreType.DMA,
        pltpu.SemaphoreType.DMA,
    ],
    compiler_params=pltpu.CompilerParams(needs_layout_passes=False),
)
def _topk_sc(x_hbm, out_hbm, buf0, buf1, out_v, sem0, sem1):
    nc = 2
    wid = lax.axis_index("s") * nc + lax.axis_index("c")
    base = wid * _RPW
    lane = lax.iota(jnp.int32, 16)
    lane_lt8 = lane < _K
    neg = jnp.full((_LANES,), -jnp.inf, jnp.float32)

    def row_compute(buf, row_local):
        def body(i, ts):
            for j in range(_UNROLL):
                v = buf[pl.ds((i * _UNROLL + j) * _LANES, _LANES)]
                ts = _insert(ts, v)
            return ts

        ts = lax.fori_loop(0, _VPR // _UNROLL, body, (neg,) * _K)
        vs = [_vsort_desc(t) for t in ts]
        while len(vs) > 1:
            vs = [_combine(vs[i], vs[i + 1], lane_lt8)
                  for i in range(0, len(vs), 2)]
        plsc.store_compressed(out_v.at[pl.ds(row_local * _K, _LANES)],
                              vs[0], mask=lane_lt8)

    pltpu.async_copy(x_hbm.at[base], buf0, sem0)
    pltpu.async_copy(x_hbm.at[base + 1], buf1, sem1)

    def step(s, carry):
        r0 = 2 * s
        pltpu.make_async_copy(x_hbm.at[base + r0], buf0, sem0).wait()
        row_compute(buf0, r0)
        nxt0 = jnp.minimum(r0 + 2, _RPW - 1)
        pltpu.async_copy(x_hbm.at[base + nxt0], buf0, sem0)

        pltpu.make_async_copy(x_hbm.at[base + r0 + 1], buf1, sem1).wait()
        row_compute(buf1, r0 + 1)
        nxt1 = jnp.minimum(r0 + 3, _RPW - 1)
        pltpu.async_copy(x_hbm.at[base + nxt1], buf1, sem1)
        return carry

    lax.fori_loop(0, _RPW // 2, step, 0)

    pltpu.make_async_copy(x_hbm.at[base], buf0, sem0).wait()
    pltpu.make_async_copy(x_hbm.at[base], buf1, sem1).wait()

    pltpu.sync_copy(out_v.at[pl.ds(0, _RPW * _K)],
                    out_hbm.at[pl.ds(base * _K, _RPW * _K)])


def kernel(x):
    out = _topk_sc(x.reshape(_ROWS, _N))
    return out.reshape(_B, _C, _K)
